# baseline (speedup 1.0000x reference)
import functools

import jax
import jax.numpy as jnp
from jax import lax
from jax.experimental import pallas as pl
from jax.experimental.pallas import tpu as pltpu
from jax.experimental.pallas import tpu_sc as plsc

L = 16
NC = 2
NS = 16
NW = NC * NS
B = 16384
D = 64
ROWW = 128
BPW = B // NW
NH = 2
HPW = BPW // NH
GCH = 128
NCH = HPW // GCH


def _gmf_body(uidx_hbm, uhalf_hbm, iidx_hbm, ihalf_hbm, utab_hbm, itab_hbm,
              w_hbm, b_hbm, out_hbm, uidx_v, uhalf_v, iidx_v, ihalf_v,
              urows_v, irows_v, w_v, b_v, out_v, gsem):
    wid = lax.axis_index("s") * NC + lax.axis_index("c")
    base = wid * BPW

    pltpu.sync_copy(uidx_hbm.at[pl.ds(base, BPW)], uidx_v)
    pltpu.sync_copy(uhalf_hbm.at[pl.ds(base, BPW)], uhalf_v)
    pltpu.sync_copy(iidx_hbm.at[pl.ds(base, BPW)], iidx_v)
    pltpu.sync_copy(ihalf_hbm.at[pl.ds(base, BPW)], ihalf_v)
    pltpu.sync_copy(w_hbm, w_v)
    pltpu.sync_copy(b_hbm, b_v)

    w0 = w_v[pl.ds(0, L)]
    w1 = w_v[pl.ds(L, L)]
    w2 = w_v[pl.ds(2 * L, L)]
    w3 = w_v[pl.ds(3 * L, L)]
    bias = b_v[...]
    lane = lax.iota(jnp.int32, L)
    perms = [lane ^ s for s in (8, 4, 2, 1)]

    def lanesum(v):
        for p in perms:
            v = v + v.at[p].get(mode="promise_in_bounds", unique_indices=True)
        return v

    for h in range(NH):
        hbase = h * HPW
        copies = []
        for c in range(NCH):
            dsl = pl.ds(c * GCH, GCH)
            ssl = pl.ds(hbase + c * GCH, GCH)
            copies.append(pltpu.async_copy(
                utab_hbm.at[uidx_v.at[ssl]], urows_v.at[dsl], gsem))
            copies.append(pltpu.async_copy(
                itab_hbm.at[iidx_v.at[ssl]], irows_v.at[dsl], gsem))
        for cp in copies:
            cp.wait()

        def group_body(j, carry):
            g = hbase + j * L
            hu16 = uhalf_v[pl.ds(g, L)]
            hi16 = ihalf_v[pl.ds(g, L)]
            res = jnp.zeros((L,), jnp.float32)
            for k in range(L):
                i = j * L + k
                bcast = jnp.full((L,), k, jnp.int32)
                bu = hu16.at[bcast].get(mode="promise_in_bounds")
                bi = hi16.at[bcast].get(mode="promise_in_bounds")
                us = [urows_v[i, pl.ds(c * L, L)] for c in range(8)]
                vs = [irows_v[i, pl.ds(c * L, L)] for c in range(8)]
                acc = jnp.zeros((L,), jnp.float32)
                for c, wc in enumerate((w0, w1, w2, w3)):
                    uc = us[c] + bu * (us[c + 4] - us[c])
                    vc = vs[c] + bi * (vs[c + 4] - vs[c])
                    acc = acc + (uc * vc) * wc
                res = jnp.where(lane == k, lanesum(acc), res)
            x = res + bias
            out_v[pl.ds(hbase + j * L, L)] = 1.0 / (1.0 + jnp.exp(-x))
            return carry

        lax.fori_loop(0, HPW // L, group_body, 0)

    pltpu.sync_copy(out_v, out_hbm.at[pl.ds(base, BPW)])


@functools.partial(jax.jit, static_argnames=())
def _gmf(uidx, uhalf, iidx, ihalf, utab2, itab2, w64, b16):
    mesh = plsc.VectorSubcoreMesh(core_axis_name="c", subcore_axis_name="s")
    run = functools.partial(
        pl.kernel,
        mesh=mesh,
        compiler_params=pltpu.CompilerParams(use_tc_tiling_on_sc=True),
        out_type=jax.ShapeDtypeStruct((B,), jnp.float32),
        scratch_types=[
            pltpu.VMEM((BPW,), jnp.int32),
            pltpu.VMEM((BPW,), jnp.float32),
            pltpu.VMEM((BPW,), jnp.int32),
            pltpu.VMEM((BPW,), jnp.float32),
            pltpu.VMEM((HPW, ROWW), jnp.float32),
            pltpu.VMEM((HPW, ROWW), jnp.float32),
            pltpu.VMEM((D,), jnp.float32),
            pltpu.VMEM((L,), jnp.float32),
            pltpu.VMEM((BPW,), jnp.float32),
            pltpu.SemaphoreType.DMA,
        ],
    )(_gmf_body)
    return run(uidx, uhalf, iidx, ihalf, utab2, itab2, w64, b16)


def kernel(user_indices, item_indices, user_table, item_table, W, b):
    ui = user_indices.astype(jnp.int32)
    ii = item_indices.astype(jnp.int32)
    utab2 = jnp.reshape(user_table, (user_table.shape[0] // 2, 2 * D))
    itab2 = jnp.reshape(item_table, (item_table.shape[0] // 2, 2 * D))
    w64 = jnp.reshape(W.astype(jnp.float32), (D,))
    b16 = jnp.full((L,), b[0], dtype=jnp.float32)
    out = _gmf(ui >> 1, (ui & 1).astype(jnp.float32),
               ii >> 1, (ii & 1).astype(jnp.float32),
               utab2, itab2, w64, b16)
    return jnp.reshape(out, (B, 1))
